# baseline (speedup 1.0000x reference)
import functools

import jax
import jax.numpy as jnp
from jax import lax
from jax.experimental import pallas as pl
from jax.experimental.pallas import tpu as pltpu
from jax.experimental.pallas import tpu_sc as plsc

B = 16384
V = 1000000
D = 64

NC = 2
NS = 16
L = 16
NW = NC * NS
BPW = B // NW
CHUNK = 128
NCH = BPW // CHUNK
PHCH = 2
PH = NCH // PHCH
PROWS = PHCH * CHUNK

_mesh = plsc.VectorSubcoreMesh(core_axis_name="c", subcore_axis_name="s")


@functools.partial(
    pl.kernel,
    out_type=jax.ShapeDtypeStruct((B,), jnp.float32),
    mesh=_mesh,
    compiler_params=pltpu.CompilerParams(
        needs_layout_passes=False, use_tc_tiling_on_sc=False),
    scratch_types=[
        pltpu.VMEM((NCH, CHUNK), jnp.int32),
        pltpu.VMEM((NCH, CHUNK), jnp.int32),
        pltpu.VMEM((NCH, CHUNK), jnp.int32),
        pltpu.VMEM((NCH, CHUNK), jnp.int32),
        pltpu.VMEM((PROWS, 2 * D), jnp.float32),
        pltpu.VMEM((PROWS, 2 * D), jnp.float32),
        pltpu.VMEM((BPW,), jnp.float32),
        pltpu.SemaphoreType.DMA,
        pltpu.SemaphoreType.DMA,
    ],
)
def _sc_scores(user_ref, item_ref, ut2_ref, it2_ref, out_ref,
               uidx, iidx, uvr, ivr, uB, iB, outv, usem, isem):
    wid = lax.axis_index("s") * NC + lax.axis_index("c")
    base = wid * BPW

    pltpu.sync_copy(user_ref.at[wid], uidx)
    pltpu.sync_copy(item_ref.at[wid], iidx)
    for j in range(NCH):
        for m in range(CHUNK // L):
            s = pl.ds(m * L, L)
            uvr[j, s] = lax.shift_right_logical(uidx[j, s], 1)
            ivr[j, s] = lax.shift_right_logical(iidx[j, s], 1)

    lane = lax.iota(jnp.int32, 16)

    for p in range(PH):
        copies = []
        for q in range(PHCH):
            j = p * PHCH + q
            dst = pl.ds(q * CHUNK, CHUNK)
            copies.append(pltpu.async_copy(ut2_ref.at[uvr.at[j]], uB.at[dst], usem))
            copies.append(pltpu.async_copy(it2_ref.at[ivr.at[j]], iB.at[dst], isem))
        for cp in copies:
            cp.wait()

        for q in range(PHCH):
            j = p * PHCH + q

            def grp_body(g, _, q=q, j=j):
                s = pl.ds(g * L, L)
                rows = q * CHUNK + g * L + lane
                upar = lax.shift_left(jnp.bitwise_and(uidx[j, s], 1), 6)
                ipar = lax.shift_left(jnp.bitwise_and(iidx[j, s], 1), 6)

                def col_body(t, carry):
                    acc, uc, ic = carry
                    for _ in range(8):
                        u = plsc.load_gather(uB, [rows, uc])
                        v = plsc.load_gather(iB, [rows, ic])
                        acc = acc + u * v
                        uc = uc + 1
                        ic = ic + 1
                    return acc, uc, ic

                acc, _, _ = lax.fori_loop(
                    0, D // 8, col_body,
                    (jnp.zeros((16,), jnp.float32), upar, ipar))
                outv[pl.ds(j * CHUNK + g * L, L)] = 1.0 / (1.0 + jnp.exp(-acc))
                return 0

            lax.fori_loop(0, CHUNK // L, grp_body, 0)

    pltpu.sync_copy(outv, out_ref.at[pl.ds(base, BPW)])


def kernel(user, item, user_table, item_table):
    user3 = user.astype(jnp.int32).reshape(NW, NCH, CHUNK)
    item3 = item.astype(jnp.int32).reshape(NW, NCH, CHUNK)
    ut2 = user_table.reshape(V // 2, 2 * D)
    it2 = item_table.reshape(V // 2, 2 * D)
    return _sc_scores(user3, item3, ut2, it2)


# speedup vs baseline: 2.1210x; 2.1210x over previous
import functools

import jax
import jax.numpy as jnp
from jax import lax
from jax.experimental import pallas as pl
from jax.experimental.pallas import tpu as pltpu
from jax.experimental.pallas import tpu_sc as plsc

B = 16384
V = 1000000
D = 64

NC = 2
NS = 16
L = 16
NW = NC * NS
BPW = B // NW
P = 32
NPH = BPW // P

_mesh = plsc.VectorSubcoreMesh(core_axis_name="c", subcore_axis_name="s")


@functools.partial(
    pl.kernel,
    out_type=jax.ShapeDtypeStruct((B,), jnp.float32),
    mesh=_mesh,
    compiler_params=pltpu.CompilerParams(
        needs_layout_passes=False, use_tc_tiling_on_sc=True),
    scratch_types=[
        pltpu.VMEM((BPW,), jnp.int32),
        pltpu.VMEM((BPW,), jnp.int32),
        pltpu.VMEM((P, 8, D), jnp.float32),
        pltpu.VMEM((P, 8, D), jnp.float32),
        pltpu.VMEM((BPW,), jnp.float32),
        pltpu.SemaphoreType.DMA,
        pltpu.SemaphoreType.DMA,
    ],
)
def _sc_scores(user_ref, item_ref, ut3_ref, it3_ref, out_ref,
               uidx, iidx, uT, iT, outv, usem, isem):
    wid = lax.axis_index("s") * NC + lax.axis_index("c")
    base = wid * BPW

    pltpu.sync_copy(user_ref.at[pl.ds(base, BPW)], uidx)
    pltpu.sync_copy(item_ref.at[pl.ds(base, BPW)], iidx)

    lane = lax.iota(jnp.int32, 16)

    def phase_body(ph, _):
        k0 = ph * P
        for g in range(P // L):
            rv = uidx[pl.ds(k0 + g * L, L)]
            sv = iidx[pl.ds(k0 + g * L, L)]
            rb = lax.shift_right_logical(rv, 3)
            sb = lax.shift_right_logical(sv, 3)
            for j in range(L):
                slot = g * L + j
                pltpu.async_copy(ut3_ref.at[rb[j]], uT.at[slot], usem)
                pltpu.async_copy(it3_ref.at[sb[j]], iT.at[slot], isem)
        pltpu.make_async_copy(ut3_ref.at[pl.ds(0, P)], uT, usem).wait()
        pltpu.make_async_copy(it3_ref.at[pl.ds(0, P)], iT, isem).wait()

        for g in range(P // L):
            s = pl.ds(k0 + g * L, L)
            slots = g * L + lane
            urow = jnp.bitwise_and(uidx[s], 7)
            irow = jnp.bitwise_and(iidx[s], 7)

            def col_body(t, acc, slots=slots, urow=urow, irow=irow):
                c = t * 8
                for dc in range(8):
                    cv = jnp.full((16,), 0, jnp.int32) + (c + dc)
                    u = plsc.load_gather(uT, [slots, urow, cv])
                    v = plsc.load_gather(iT, [slots, irow, cv])
                    acc = acc + u * v
                return acc

            acc = lax.fori_loop(0, D // 8, col_body,
                                jnp.zeros((16,), jnp.float32))
            outv[pl.ds(k0 + g * L, L)] = 1.0 / (1.0 + jnp.exp(-acc))
        return 0

    lax.fori_loop(0, NPH, phase_body, 0)
    pltpu.sync_copy(outv, out_ref.at[pl.ds(base, BPW)])


def kernel(user, item, user_table, item_table):
    ut3 = user_table.reshape(V // 8, 8, D)
    it3 = item_table.reshape(V // 8, 8, D)
    return _sc_scores(user.astype(jnp.int32), item.astype(jnp.int32), ut3, it3)


# speedup vs baseline: 2.2560x; 1.0637x over previous
import functools

import jax
import jax.numpy as jnp
from jax import lax
from jax.experimental import pallas as pl
from jax.experimental.pallas import tpu as pltpu
from jax.experimental.pallas import tpu_sc as plsc

B = 16384
V = 1000000
D = 64

NC = 2
NS = 16
L = 16
NW = NC * NS
BPW = B // NW
P = 16
NPH = BPW // P

_mesh = plsc.VectorSubcoreMesh(core_axis_name="c", subcore_axis_name="s")


@functools.partial(
    pl.kernel,
    out_type=jax.ShapeDtypeStruct((B,), jnp.float32),
    mesh=_mesh,
    compiler_params=pltpu.CompilerParams(
        needs_layout_passes=False, use_tc_tiling_on_sc=True),
    scratch_types=[
        pltpu.VMEM((BPW,), jnp.int32),
        pltpu.VMEM((BPW,), jnp.int32),
        pltpu.VMEM((BPW,), jnp.int32),
        pltpu.VMEM((BPW,), jnp.int32),
        pltpu.VMEM((P, 8, D), jnp.float32),
        pltpu.VMEM((P, 8, D), jnp.float32),
        pltpu.VMEM((P, 8, D), jnp.float32),
        pltpu.VMEM((P, 8, D), jnp.float32),
        pltpu.VMEM((BPW,), jnp.float32),
        pltpu.SemaphoreType.DMA,
        pltpu.SemaphoreType.DMA,
        pltpu.SemaphoreType.DMA,
        pltpu.SemaphoreType.DMA,
    ],
)
def _sc_scores(user_ref, item_ref, ut3_ref, it3_ref, out_ref,
               uidx, iidx, ublk, iblk, uT0, uT1, iT0, iT1, outv,
               su0, su1, si0, si1):
    wid = lax.axis_index("s") * NC + lax.axis_index("c")
    base = wid * BPW

    pltpu.sync_copy(user_ref.at[pl.ds(base, BPW)], uidx)
    pltpu.sync_copy(item_ref.at[pl.ds(base, BPW)], iidx)
    for m in range(BPW // L):
        s = pl.ds(m * L, L)
        ublk[s] = lax.shift_right_logical(uidx[s], 3)
        iblk[s] = lax.shift_right_logical(iidx[s], 3)

    lane = lax.iota(jnp.int32, 16)

    def issue(ph, uT, iT, su, si):
        s = pl.ds(ph * P, P)
        rb = ublk[s]
        sb = iblk[s]
        for j in range(P):
            pltpu.async_copy(ut3_ref.at[rb[j]], uT.at[j], su)
            pltpu.async_copy(it3_ref.at[sb[j]], iT.at[j], si)

    def drain(uT, iT, su, si):
        pltpu.make_async_copy(ut3_ref.at[pl.ds(0, P)], uT, su).wait()
        pltpu.make_async_copy(it3_ref.at[pl.ds(0, P)], iT, si).wait()

    def compute(ph, uT, iT):
        s = pl.ds(ph * P, P)
        urow = jnp.bitwise_and(uidx[s], 7)
        irow = jnp.bitwise_and(iidx[s], 7)

        def col_body(t, acc):
            c = t * 8
            for dc in range(8):
                cv = jnp.full((16,), c + dc, jnp.int32)
                u = plsc.load_gather(uT, [lane, urow, cv])
                v = plsc.load_gather(iT, [lane, irow, cv])
                acc = acc + u * v
            return acc

        acc = lax.fori_loop(0, D // 8, col_body, jnp.zeros((16,), jnp.float32))
        outv[s] = 1.0 / (1.0 + jnp.exp(-acc))

    issue(0, uT0, iT0, su0, si0)

    def pair_body(q, _):
        p0 = 2 * q
        issue(p0 + 1, uT1, iT1, su1, si1)
        drain(uT0, iT0, su0, si0)
        compute(p0, uT0, iT0)

        @pl.when(q < NPH // 2 - 1)
        def _():
            issue(p0 + 2, uT0, iT0, su0, si0)

        drain(uT1, iT1, su1, si1)
        compute(p0 + 1, uT1, iT1)
        return 0

    lax.fori_loop(0, NPH // 2, pair_body, 0)
    pltpu.sync_copy(outv, out_ref.at[pl.ds(base, BPW)])


def kernel(user, item, user_table, item_table):
    ut3 = user_table.reshape(V // 8, 8, D)
    it3 = item_table.reshape(V // 8, 8, D)
    return _sc_scores(user.astype(jnp.int32), item.astype(jnp.int32), ut3, it3)
